# baseline (speedup 1.0000x reference)
import functools

import jax
import jax.numpy as jnp
from jax import lax
from jax.experimental import pallas as pl
from jax.experimental.pallas import tpu as pltpu
from jax.experimental.pallas import tpu_sc as plsc

N, E = 10000, 160000
D_IN, D_INNER, D_OUT = 256, 512, 256
NC, NS = 2, 16
NW = NC * NS
BUCKET = 320
GARBAGE = BUCKET
CNT_ROW = BUCKET + 1
ACC_R = 328
STAGE = 1024
E_PAD = 163840
FLUSH = 128
DUMP = FLUSH + 16
RB = 1000


def _mlp_body(x_ref, w1_ref, w2_ref, o_ref):
    h = jnp.maximum(
        jnp.dot(x_ref[...], w1_ref[...], preferred_element_type=jnp.float32), 0.0)
    o_ref[...] = jnp.dot(h, w2_ref[...], preferred_element_type=jnp.float32)


def _node_mlp(xx, w1, w2):
    return pl.pallas_call(
        _mlp_body,
        grid=(N // RB,),
        in_specs=[
            pl.BlockSpec((RB, D_OUT), lambda i: (i, 0)),
            pl.BlockSpec((D_OUT, D_INNER), lambda i: (0, 0)),
            pl.BlockSpec((D_INNER, D_OUT), lambda i: (0, 0)),
        ],
        out_specs=pl.BlockSpec((RB, D_OUT), lambda i: (i, 0)),
        out_shape=jax.ShapeDtypeStruct((N, D_OUT), jnp.float32),
    )(xx, w1, w2)


def _mid_body(s_ref, c_ref, x_ref, wi_ref, bi_ref, w1_ref, w2_ref, g_ref, b_ref,
              h2_ref, m2_ref):
    cnt = c_ref[...]
    mean = s_ref[...] / jnp.maximum(cnt, 1.0)
    h = jnp.where(cnt > 0.0, mean, x_ref[...])
    mu = jnp.mean(h, axis=-1, keepdims=True)
    var = jnp.mean((h - mu) ** 2, axis=-1, keepdims=True)
    h = (h - mu) * lax.rsqrt(var + 1e-5) * g_ref[...] + b_ref[...]
    h2 = jnp.maximum(
        jnp.dot(h, wi_ref[...], preferred_element_type=jnp.float32) + bi_ref[...],
        0.0)
    h2_ref[...] = h2
    m2_ref[...] = jnp.dot(
        jnp.maximum(jnp.dot(h2, w1_ref[...], preferred_element_type=jnp.float32),
                    0.0),
        w2_ref[...], preferred_element_type=jnp.float32)


def _mid_stage(s1, cnt_b, x, wi, bi2, w1b, w2b, g1, b1):
    blk = pl.BlockSpec((RB, D_OUT), lambda i: (i, 0))
    full = lambda shape: pl.BlockSpec(shape, lambda i: (0,) * len(shape))
    return pl.pallas_call(
        _mid_body,
        grid=(N // RB,),
        in_specs=[
            blk, blk, blk,
            full((D_OUT, D_OUT)), full((1, D_OUT)),
            full((D_OUT, D_INNER)), full((D_INNER, D_OUT)),
            full((1, D_OUT)), full((1, D_OUT)),
        ],
        out_specs=[blk, blk],
        out_shape=[jax.ShapeDtypeStruct((N, D_OUT), jnp.float32),
                   jax.ShapeDtypeStruct((N, D_OUT), jnp.float32)],
    )(s1, cnt_b, x, wi, bi2, w1b, w2b, g1, b1)


def _final_body(s_ref, c_ref, h_ref, g_ref, b_ref, o_ref):
    cnt = c_ref[...]
    mean = s_ref[...] / jnp.maximum(cnt, 1.0)
    h = jnp.where(cnt > 0.0, mean, h_ref[...])
    mu = jnp.mean(h, axis=-1, keepdims=True)
    var = jnp.mean((h - mu) ** 2, axis=-1, keepdims=True)
    o_ref[...] = (h - mu) * lax.rsqrt(var + 1e-5) * g_ref[...] + b_ref[...]


def _final_stage(s2, cnt_b, h2, g2, b2):
    blk = pl.BlockSpec((RB, D_OUT), lambda i: (i, 0))
    full = lambda shape: pl.BlockSpec(shape, lambda i: (0,) * len(shape))
    return pl.pallas_call(
        _final_body,
        grid=(N // RB,),
        in_specs=[blk, blk, blk, full((1, D_OUT)), full((1, D_OUT))],
        out_specs=blk,
        out_shape=jax.ShapeDtypeStruct((N, D_OUT), jnp.float32),
    )(s2, cnt_b, h2, g2, b2)


@functools.cache
def _make_segsum():
    mesh = plsc.VectorSubcoreMesh(core_axis_name="c", subcore_axis_name="s")
    out_type = jax.ShapeDtypeStruct((NC, NS, ACC_R, D_OUT), jnp.float32)
    scratch = [
        pltpu.VMEM((ACC_R, D_OUT), jnp.float32),
        pltpu.VMEM((BUCKET + 1, 16), jnp.float32),
        pltpu.VMEM((FLUSH, D_OUT), jnp.float32),
        pltpu.VMEM((STAGE,), jnp.int32),
        pltpu.VMEM((STAGE,), jnp.int32),
        pltpu.VMEM((STAGE,), jnp.int32),
        pltpu.VMEM((STAGE,), jnp.int32),
        pltpu.VMEM((2 * FLUSH,), jnp.int32),
        pltpu.VMEM((2 * FLUSH,), jnp.int32),
        pltpu.SemaphoreType.DMA,
        pltpu.SemaphoreType.DMA,
        pltpu.SemaphoreType.DMA,
    ]

    def body(m_hbm, src_hbm, dst_hbm, out_hbm, acc, cacc, grow,
             dstage0, sstage0, dstage1, sstage1, srcc, rowc, sem,
             stsem0, stsem1):
        c = lax.axis_index("c")
        s = lax.axis_index("s")
        lo = (c * NS + s) * BUCKET
        zf = jnp.zeros((16,), jnp.float32)
        iota16 = lax.iota(jnp.int32, 16)
        onev = jnp.where(iota16 == 0, jnp.float32(1.0), jnp.float32(0.0))

        def zr(r, _):
            def zc(l, _):
                acc[r, pl.ds(l * 16, 16)] = zf
                return 0
            return lax.fori_loop(0, D_OUT // 16, zc, 0)
        lax.fori_loop(0, ACC_R, zr, 0)

        def zo(r, _):
            cacc[r, :] = zf
            return 0
        lax.fori_loop(0, BUCKET + 1, zo, 0)

        def flush():
            pltpu.async_copy(m_hbm.at[srcc.at[pl.ds(0, FLUSH)]], grow,
                             sem).wait()

            def grp(g, _):
                rv = rowc[pl.ds(g * 16, 16)]
                for e in range(16):
                    r = rv[e]
                    ge = g * 16 + e
                    for l in range(D_OUT // 16):
                        plsc.addupdate(acc.at[r, pl.ds(l * 16, 16)],
                                       grow[ge, pl.ds(l * 16, 16)])
                    plsc.addupdate(cacc.at[r], onev)
                return 0
            lax.fori_loop(0, FLUSH // 16, grp, 0)
            srcc[pl.ds(0, 16)] = srcc[pl.ds(FLUSH, 16)]
            rowc[pl.ds(0, 16)] = rowc[pl.ds(FLUSH, 16)]

        NSTAGE = E_PAD // STAGE
        bufs = ((dstage0, sstage0, stsem0), (dstage1, sstage1, stsem1))

        def issue(t, b):
            db, sb, sm = bufs[b]
            pltpu.async_copy(dst_hbm.at[pl.ds(t * STAGE, STAGE)], db, sm)
            pltpu.async_copy(src_hbm.at[pl.ds(t * STAGE, STAGE)], sb, sm)

        issue(0, 0)
        issue(1, 1)

        def make_step(b):
            dstage, sstage, stsem = bufs[b]

            def stage_step(t, off):
                pltpu.make_async_copy(
                    dst_hbm.at[pl.ds(t * STAGE, STAGE)], dstage, stsem).wait()
                pltpu.make_async_copy(
                    src_hbm.at[pl.ds(t * STAGE, STAGE)], sstage, stsem).wait()

                def group(g, off):
                    d = dstage[pl.ds(g * 16, 16)]
                    sv = sstage[pl.ds(g * 16, 16)]
                    rel = d - lo
                    ok = (rel >= 0) & (rel < BUCKET)
                    pc = plsc.all_reduce_population_count(ok)

                    def do_compact():
                        oki = jnp.where(ok, 1, 0)
                        inc = plsc.cumsum(oki)
                        pos = jnp.where(ok, off + inc - oki, DUMP + iota16)
                        plsc.store_scatter(srcc, [pos], sv)
                        plsc.store_scatter(rowc, [pos], rel)
                        return off + pc[0]
                    off = lax.cond(pc[0] > 0, do_compact, lambda: off)

                    def do_flush():
                        flush()
                        return off - FLUSH
                    return lax.cond(off >= FLUSH, do_flush, lambda: off)
                off = lax.fori_loop(0, STAGE // 16, group, off)
                nxt = t + 2

                @pl.when(nxt < NSTAGE)
                def _():
                    issue(nxt, b)
                return off
            return stage_step

        step_for = (make_step(0), make_step(1))

        def pair_step(jo, off):
            for b in range(2):
                off = step_for[b](jo * 2 + b, off)
            return off

        off = lax.fori_loop(0, NSTAGE // 2, pair_step, 0)

        zi = jnp.zeros((16,), jnp.int32)
        gv = jnp.full((16,), GARBAGE, jnp.int32)
        for k in range(FLUSH // 16):
            srcc[pl.ds(off + k * 16, 16)] = zi
            rowc[pl.ds(off + k * 16, 16)] = gv
        flush()

        for j in range(BUCKET // 16):
            cv = plsc.load_gather(cacc, [iota16 + j * 16, zi])
            acc[CNT_ROW + (j * 16) // D_OUT,
                pl.ds((j * 16) % D_OUT, 16)] = cv.astype(jnp.float32)

        pltpu.sync_copy(acc, out_hbm.at[c, s])

    return pl.kernel(body, out_type=out_type, mesh=mesh, scratch_types=scratch,
                     compiler_params=pltpu.CompilerParams(
                         needs_layout_passes=False,
                         use_tc_tiling_on_sc=False))


def kernel(x, edge_index, W1a, W2a, ln1_g, ln1_b, Wi, bi, W1b, W2b, ln2_g, ln2_b):
    src = edge_index[0].astype(jnp.int32)
    dst = edge_index[1].astype(jnp.int32)
    pad = E_PAD - E
    src_f = jnp.concatenate([src, jnp.zeros((pad,), jnp.int32)])
    dst_f = jnp.concatenate([dst, jnp.full((pad,), NW * BUCKET, jnp.int32)])

    bi2 = bi.reshape(1, D_OUT)
    g1 = ln1_g.reshape(1, D_OUT)
    b1 = ln1_b.reshape(1, D_OUT)
    g2 = ln2_g.reshape(1, D_OUT)
    b2 = ln2_b.reshape(1, D_OUT)

    def unpack(o):
        flat = o.reshape(NW, ACC_R * D_OUT)
        sums = flat[:, :BUCKET * D_OUT].reshape(NW * BUCKET, D_OUT)[:N]
        cnt = flat[:, CNT_ROW * D_OUT:CNT_ROW * D_OUT + BUCKET]
        cnt = cnt.reshape(NW * BUCKET)[:N]
        return sums, cnt

    m1 = _node_mlp(x, W1a, W2a)
    s1, cnt = unpack(_make_segsum()(m1, src_f, dst_f))
    cnt_b = jnp.broadcast_to(cnt[:, None], (N, D_OUT))
    h2, m2 = _mid_stage(s1, cnt_b, x, Wi, bi2, W1b, W2b, g1, b1)
    s2, _ = unpack(_make_segsum()(m2, src_f, dst_f))
    return _final_stage(s2, cnt_b, h2, g2, b2)


# speedup vs baseline: 1.1286x; 1.1286x over previous
import functools

import jax
import jax.numpy as jnp
from jax import lax
from jax.experimental import pallas as pl
from jax.experimental.pallas import tpu as pltpu
from jax.experimental.pallas import tpu_sc as plsc

N, E = 10000, 160000
D_IN, D_INNER, D_OUT = 256, 512, 256
NC, NS = 2, 16
NW = NC * NS
BUCKET = 320
GARBAGE = BUCKET
CNT_ROW = BUCKET + 1
ACC_R = 328
STAGE = 1024
E_PAD = 163840
FLUSH = 64
DUMP = FLUSH + 16
RB = 1000


def _mlp_body(x_ref, w1_ref, w2_ref, o_ref):
    h = jnp.maximum(
        jnp.dot(x_ref[...], w1_ref[...], preferred_element_type=jnp.float32), 0.0)
    o_ref[...] = jnp.dot(h, w2_ref[...], preferred_element_type=jnp.float32)


def _node_mlp(xx, w1, w2):
    return pl.pallas_call(
        _mlp_body,
        grid=(N // RB,),
        in_specs=[
            pl.BlockSpec((RB, D_OUT), lambda i: (i, 0)),
            pl.BlockSpec((D_OUT, D_INNER), lambda i: (0, 0)),
            pl.BlockSpec((D_INNER, D_OUT), lambda i: (0, 0)),
        ],
        out_specs=pl.BlockSpec((RB, D_OUT), lambda i: (i, 0)),
        out_shape=jax.ShapeDtypeStruct((N, D_OUT), jnp.float32),
    )(xx, w1, w2)


def _mid_body(s_ref, c_ref, x_ref, wi_ref, bi_ref, w1_ref, w2_ref, g_ref, b_ref,
              h2_ref, m2_ref):
    cnt = c_ref[...]
    mean = s_ref[...] / jnp.maximum(cnt, 1.0)
    h = jnp.where(cnt > 0.0, mean, x_ref[...])
    mu = jnp.mean(h, axis=-1, keepdims=True)
    var = jnp.mean((h - mu) ** 2, axis=-1, keepdims=True)
    h = (h - mu) * lax.rsqrt(var + 1e-5) * g_ref[...] + b_ref[...]
    h2 = jnp.maximum(
        jnp.dot(h, wi_ref[...], preferred_element_type=jnp.float32) + bi_ref[...],
        0.0)
    h2_ref[...] = h2
    m2_ref[...] = jnp.dot(
        jnp.maximum(jnp.dot(h2, w1_ref[...], preferred_element_type=jnp.float32),
                    0.0),
        w2_ref[...], preferred_element_type=jnp.float32)


def _mid_stage(s1, cnt_b, x, wi, bi2, w1b, w2b, g1, b1):
    blk = pl.BlockSpec((RB, D_OUT), lambda i: (i, 0))
    full = lambda shape: pl.BlockSpec(shape, lambda i: (0,) * len(shape))
    return pl.pallas_call(
        _mid_body,
        grid=(N // RB,),
        in_specs=[
            blk, blk, blk,
            full((D_OUT, D_OUT)), full((1, D_OUT)),
            full((D_OUT, D_INNER)), full((D_INNER, D_OUT)),
            full((1, D_OUT)), full((1, D_OUT)),
        ],
        out_specs=[blk, blk],
        out_shape=[jax.ShapeDtypeStruct((N, D_OUT), jnp.float32),
                   jax.ShapeDtypeStruct((N, D_OUT), jnp.float32)],
    )(s1, cnt_b, x, wi, bi2, w1b, w2b, g1, b1)


def _final_body(s_ref, c_ref, h_ref, g_ref, b_ref, o_ref):
    cnt = c_ref[...]
    mean = s_ref[...] / jnp.maximum(cnt, 1.0)
    h = jnp.where(cnt > 0.0, mean, h_ref[...])
    mu = jnp.mean(h, axis=-1, keepdims=True)
    var = jnp.mean((h - mu) ** 2, axis=-1, keepdims=True)
    o_ref[...] = (h - mu) * lax.rsqrt(var + 1e-5) * g_ref[...] + b_ref[...]


def _final_stage(s2, cnt_b, h2, g2, b2):
    blk = pl.BlockSpec((RB, D_OUT), lambda i: (i, 0))
    full = lambda shape: pl.BlockSpec(shape, lambda i: (0,) * len(shape))
    return pl.pallas_call(
        _final_body,
        grid=(N // RB,),
        in_specs=[blk, blk, blk, full((1, D_OUT)), full((1, D_OUT))],
        out_specs=blk,
        out_shape=jax.ShapeDtypeStruct((N, D_OUT), jnp.float32),
    )(s2, cnt_b, h2, g2, b2)


@functools.cache
def _make_segsum():
    mesh = plsc.VectorSubcoreMesh(core_axis_name="c", subcore_axis_name="s")
    out_type = jax.ShapeDtypeStruct((NC, NS, ACC_R, D_OUT), jnp.float32)
    scratch = [
        pltpu.VMEM((ACC_R, D_OUT), jnp.float32),
        pltpu.VMEM((BUCKET + 1, 16), jnp.float32),
        pltpu.VMEM((FLUSH, D_OUT), jnp.float32),
        pltpu.VMEM((FLUSH, D_OUT), jnp.float32),
        pltpu.VMEM((STAGE,), jnp.int32),
        pltpu.VMEM((STAGE,), jnp.int32),
        pltpu.VMEM((STAGE,), jnp.int32),
        pltpu.VMEM((STAGE,), jnp.int32),
        pltpu.VMEM((2 * FLUSH + 16,), jnp.int32),
        pltpu.VMEM((2 * FLUSH + 16,), jnp.int32),
        pltpu.VMEM((FLUSH,), jnp.int32),
        pltpu.VMEM((FLUSH,), jnp.int32),
        pltpu.VMEM((FLUSH,), jnp.int32),
        pltpu.VMEM((FLUSH,), jnp.int32),
        pltpu.SemaphoreType.DMA,
        pltpu.SemaphoreType.DMA,
        pltpu.SemaphoreType.DMA,
        pltpu.SemaphoreType.DMA,
    ]

    def body(m_hbm, src_hbm, dst_hbm, out_hbm, acc, cacc, grow0, grow1,
             dstage0, sstage0, dstage1, sstage1, srcc, rowc,
             gsrc0, gsrc1, growr0, growr1, sem0, sem1, stsem0, stsem1):
        c = lax.axis_index("c")
        s = lax.axis_index("s")
        lo = (c * NS + s) * BUCKET
        zf = jnp.zeros((16,), jnp.float32)
        iota16 = lax.iota(jnp.int32, 16)
        onev = jnp.where(iota16 == 0, jnp.float32(1.0), jnp.float32(0.0))

        def zr(r, _):
            def zc(l, _):
                acc[r, pl.ds(l * 16, 16)] = zf
                return 0
            return lax.fori_loop(0, D_OUT // 16, zc, 0)
        lax.fori_loop(0, ACC_R, zr, 0)

        def zo(r, _):
            cacc[r, :] = zf
            return 0
        lax.fori_loop(0, BUCKET + 1, zo, 0)

        par = ((grow0, gsrc0, growr0, sem0), (grow1, gsrc1, growr1, sem1))

        def snap_and_fire(p):
            growb, gsrcb, growrb, semb = par[p]
            for q in range(FLUSH // 16):
                gsrcb[pl.ds(q * 16, 16)] = srcc[pl.ds(q * 16, 16)]
                growrb[pl.ds(q * 16, 16)] = rowc[pl.ds(q * 16, 16)]
            pltpu.async_copy(m_hbm.at[gsrcb.at[pl.ds(0, FLUSH)]], growb, semb)

        def acc_flush(p):
            growb, gsrcb, growrb, semb = par[p]
            pltpu.make_async_copy(m_hbm.at[gsrcb.at[pl.ds(0, FLUSH)]], growb,
                                  semb).wait()

            def grp(g, _):
                rv = growrb[pl.ds(g * 16, 16)]
                for e in range(16):
                    r = rv[e]
                    ge = g * 16 + e
                    for l in range(D_OUT // 16):
                        plsc.addupdate(acc.at[r, pl.ds(l * 16, 16)],
                                       growb[ge, pl.ds(l * 16, 16)])
                    plsc.addupdate(cacc.at[r], onev)
                return 0
            lax.fori_loop(0, FLUSH // 16, grp, 0)

        def flush(fk):
            @pl.when(fk >= 1)
            def _():
                lax.cond((fk & 1) == 1, lambda: acc_flush(0),
                         lambda: acc_flush(1))
            lax.cond((fk & 1) == 0, lambda: snap_and_fire(0),
                     lambda: snap_and_fire(1))
            srcc[pl.ds(0, 16)] = srcc[pl.ds(FLUSH, 16)]
            rowc[pl.ds(0, 16)] = rowc[pl.ds(FLUSH, 16)]

        NSTAGE = E_PAD // STAGE
        bufs = ((dstage0, sstage0, stsem0), (dstage1, sstage1, stsem1))

        def issue(t, b):
            db, sb, sm = bufs[b]
            pltpu.async_copy(dst_hbm.at[pl.ds(t * STAGE, STAGE)], db, sm)
            pltpu.async_copy(src_hbm.at[pl.ds(t * STAGE, STAGE)], sb, sm)

        issue(0, 0)
        issue(1, 1)

        def make_step(b):
            dstage, sstage, stsem = bufs[b]

            def stage_step(t, carry):
                pltpu.make_async_copy(
                    dst_hbm.at[pl.ds(t * STAGE, STAGE)], dstage, stsem).wait()
                pltpu.make_async_copy(
                    src_hbm.at[pl.ds(t * STAGE, STAGE)], sstage, stsem).wait()

                def group(g, carry):
                    off, fk = carry
                    d = dstage[pl.ds(g * 16, 16)]
                    sv = sstage[pl.ds(g * 16, 16)]
                    rel = d - lo
                    ok = (rel >= 0) & (rel < BUCKET)
                    pc = plsc.all_reduce_population_count(ok)

                    def do_compact():
                        oki = jnp.where(ok, 1, 0)
                        inc = plsc.cumsum(oki)
                        pos = jnp.where(ok, off + inc - oki, DUMP + iota16)
                        plsc.store_scatter(srcc, [pos], sv)
                        plsc.store_scatter(rowc, [pos], rel)
                        return off + pc[0]
                    off = lax.cond(pc[0] > 0, do_compact, lambda: off)

                    def do_flush():
                        flush(fk)
                        return off - FLUSH, fk + 1
                    return lax.cond(off >= FLUSH, do_flush, lambda: (off, fk))
                carry = lax.fori_loop(0, STAGE // 16, group, carry)
                nxt = t + 2

                @pl.when(nxt < NSTAGE)
                def _():
                    issue(nxt, b)
                return carry
            return stage_step

        step_for = (make_step(0), make_step(1))

        def pair_step(jo, carry):
            for b in range(2):
                carry = step_for[b](jo * 2 + b, carry)
            return carry

        off, fk = lax.fori_loop(0, NSTAGE // 2, pair_step, (0, 0))

        zi = jnp.zeros((16,), jnp.int32)
        gv = jnp.full((16,), GARBAGE, jnp.int32)
        for k in range(FLUSH // 16):
            srcc[pl.ds(off + k * 16, 16)] = zi
            rowc[pl.ds(off + k * 16, 16)] = gv
        flush(fk)
        fk = fk + 1
        lax.cond((fk & 1) == 1, lambda: acc_flush(0), lambda: acc_flush(1))

        for j in range(BUCKET // 16):
            cv = plsc.load_gather(cacc, [iota16 + j * 16, zi])
            acc[CNT_ROW + (j * 16) // D_OUT,
                pl.ds((j * 16) % D_OUT, 16)] = cv.astype(jnp.float32)

        pltpu.sync_copy(acc, out_hbm.at[c, s])

    return pl.kernel(body, out_type=out_type, mesh=mesh, scratch_types=scratch,
                     compiler_params=pltpu.CompilerParams(
                         needs_layout_passes=False,
                         use_tc_tiling_on_sc=False))


def kernel(x, edge_index, W1a, W2a, ln1_g, ln1_b, Wi, bi, W1b, W2b, ln2_g, ln2_b):
    src = edge_index[0].astype(jnp.int32)
    dst = edge_index[1].astype(jnp.int32)
    pad = E_PAD - E
    src_f = jnp.concatenate([src, jnp.zeros((pad,), jnp.int32)])
    dst_f = jnp.concatenate([dst, jnp.full((pad,), NW * BUCKET, jnp.int32)])

    bi2 = bi.reshape(1, D_OUT)
    g1 = ln1_g.reshape(1, D_OUT)
    b1 = ln1_b.reshape(1, D_OUT)
    g2 = ln2_g.reshape(1, D_OUT)
    b2 = ln2_b.reshape(1, D_OUT)

    def unpack(o):
        flat = o.reshape(NW, ACC_R * D_OUT)
        sums = flat[:, :BUCKET * D_OUT].reshape(NW * BUCKET, D_OUT)[:N]
        cnt = flat[:, CNT_ROW * D_OUT:CNT_ROW * D_OUT + BUCKET]
        cnt = cnt.reshape(NW * BUCKET)[:N]
        return sums, cnt

    m1 = _node_mlp(x, W1a, W2a)
    s1, cnt = unpack(_make_segsum()(m1, src_f, dst_f))
    cnt_b = jnp.broadcast_to(cnt[:, None], (N, D_OUT))
    h2, m2 = _mid_stage(s1, cnt_b, x, Wi, bi2, W1b, W2b, g1, b1)
    s2, _ = unpack(_make_segsum()(m2, src_f, dst_f))
    return _final_stage(s2, cnt_b, h2, g2, b2)


# speedup vs baseline: 1.3705x; 1.2144x over previous
import functools

import jax
import jax.numpy as jnp
from jax import lax
from jax.experimental import pallas as pl
from jax.experimental.pallas import tpu as pltpu
from jax.experimental.pallas import tpu_sc as plsc

N, E = 10000, 160000
D_IN, D_INNER, D_OUT = 256, 512, 256
NC, NS = 2, 16
NW = NC * NS
BUCKET = 320
GARBAGE = BUCKET
CNT_ROW = BUCKET + 1
ACC_R = 328
STAGE = 1024
E_PAD = 163840
FLUSH = 64
DUMP = FLUSH + 32
RB = 1000


def _mlp_body(x_ref, w1_ref, w2_ref, o_ref):
    h = jnp.maximum(
        jnp.dot(x_ref[...], w1_ref[...], preferred_element_type=jnp.float32), 0.0)
    o_ref[...] = jnp.dot(h, w2_ref[...], preferred_element_type=jnp.float32)


def _node_mlp(xx, w1, w2):
    return pl.pallas_call(
        _mlp_body,
        grid=(N // RB,),
        in_specs=[
            pl.BlockSpec((RB, D_OUT), lambda i: (i, 0)),
            pl.BlockSpec((D_OUT, D_INNER), lambda i: (0, 0)),
            pl.BlockSpec((D_INNER, D_OUT), lambda i: (0, 0)),
        ],
        out_specs=pl.BlockSpec((RB, D_OUT), lambda i: (i, 0)),
        out_shape=jax.ShapeDtypeStruct((N, D_OUT), jnp.float32),
    )(xx, w1, w2)


def _mid_body(s_ref, c_ref, x_ref, wi_ref, bi_ref, w1_ref, w2_ref, g_ref, b_ref,
              h2_ref, m2_ref):
    cnt = c_ref[...]
    mean = s_ref[...] / jnp.maximum(cnt, 1.0)
    h = jnp.where(cnt > 0.0, mean, x_ref[...])
    mu = jnp.mean(h, axis=-1, keepdims=True)
    var = jnp.mean((h - mu) ** 2, axis=-1, keepdims=True)
    h = (h - mu) * lax.rsqrt(var + 1e-5) * g_ref[...] + b_ref[...]
    h2 = jnp.maximum(
        jnp.dot(h, wi_ref[...], preferred_element_type=jnp.float32) + bi_ref[...],
        0.0)
    h2_ref[...] = h2
    m2_ref[...] = jnp.dot(
        jnp.maximum(jnp.dot(h2, w1_ref[...], preferred_element_type=jnp.float32),
                    0.0),
        w2_ref[...], preferred_element_type=jnp.float32)


def _mid_stage(s1, cnt_b, x, wi, bi2, w1b, w2b, g1, b1):
    blk = pl.BlockSpec((RB, D_OUT), lambda i: (i, 0))
    full = lambda shape: pl.BlockSpec(shape, lambda i: (0,) * len(shape))
    return pl.pallas_call(
        _mid_body,
        grid=(N // RB,),
        in_specs=[
            blk, blk, blk,
            full((D_OUT, D_OUT)), full((1, D_OUT)),
            full((D_OUT, D_INNER)), full((D_INNER, D_OUT)),
            full((1, D_OUT)), full((1, D_OUT)),
        ],
        out_specs=[blk, blk],
        out_shape=[jax.ShapeDtypeStruct((N, D_OUT), jnp.float32),
                   jax.ShapeDtypeStruct((N, D_OUT), jnp.float32)],
    )(s1, cnt_b, x, wi, bi2, w1b, w2b, g1, b1)


def _final_body(s_ref, c_ref, h_ref, g_ref, b_ref, o_ref):
    cnt = c_ref[...]
    mean = s_ref[...] / jnp.maximum(cnt, 1.0)
    h = jnp.where(cnt > 0.0, mean, h_ref[...])
    mu = jnp.mean(h, axis=-1, keepdims=True)
    var = jnp.mean((h - mu) ** 2, axis=-1, keepdims=True)
    o_ref[...] = (h - mu) * lax.rsqrt(var + 1e-5) * g_ref[...] + b_ref[...]


def _final_stage(s2, cnt_b, h2, g2, b2):
    blk = pl.BlockSpec((RB, D_OUT), lambda i: (i, 0))
    full = lambda shape: pl.BlockSpec(shape, lambda i: (0,) * len(shape))
    return pl.pallas_call(
        _final_body,
        grid=(N // RB,),
        in_specs=[blk, blk, blk, full((1, D_OUT)), full((1, D_OUT))],
        out_specs=blk,
        out_shape=jax.ShapeDtypeStruct((N, D_OUT), jnp.float32),
    )(s2, cnt_b, h2, g2, b2)


@functools.cache
def _make_segsum():
    mesh = plsc.VectorSubcoreMesh(core_axis_name="c", subcore_axis_name="s")
    out_type = jax.ShapeDtypeStruct((NC, NS, ACC_R, D_OUT), jnp.float32)
    scratch = [
        pltpu.VMEM((ACC_R, D_OUT), jnp.float32),
        pltpu.VMEM((BUCKET + 1, 16), jnp.float32),
        pltpu.VMEM((FLUSH, D_OUT), jnp.float32),
        pltpu.VMEM((FLUSH, D_OUT), jnp.float32),
        pltpu.VMEM((STAGE,), jnp.int32),
        pltpu.VMEM((STAGE,), jnp.int32),
        pltpu.VMEM((STAGE,), jnp.int32),
        pltpu.VMEM((STAGE,), jnp.int32),
        pltpu.VMEM((2 * FLUSH + 16,), jnp.int32),
        pltpu.VMEM((2 * FLUSH + 16,), jnp.int32),
        pltpu.VMEM((FLUSH,), jnp.int32),
        pltpu.VMEM((FLUSH,), jnp.int32),
        pltpu.VMEM((FLUSH,), jnp.int32),
        pltpu.VMEM((FLUSH,), jnp.int32),
        pltpu.SemaphoreType.DMA,
        pltpu.SemaphoreType.DMA,
        pltpu.SemaphoreType.DMA,
        pltpu.SemaphoreType.DMA,
    ]

    def body(m_hbm, src_hbm, dst_hbm, out_hbm, acc, cacc, grow0, grow1,
             dstage0, sstage0, dstage1, sstage1, srcc, rowc,
             gsrc0, gsrc1, growr0, growr1, sem0, sem1, stsem0, stsem1):
        c = lax.axis_index("c")
        s = lax.axis_index("s")
        lo = (c * NS + s) * BUCKET
        zf = jnp.zeros((16,), jnp.float32)
        iota16 = lax.iota(jnp.int32, 16)
        onev = jnp.where(iota16 == 0, jnp.float32(1.0), jnp.float32(0.0))

        def zr(r, _):
            def zc(l, _):
                acc[r, pl.ds(l * 16, 16)] = zf
                return 0
            return lax.fori_loop(0, D_OUT // 16, zc, 0)
        lax.fori_loop(0, ACC_R, zr, 0)

        def zo(r, _):
            cacc[r, :] = zf
            return 0
        lax.fori_loop(0, BUCKET + 1, zo, 0)

        par = ((grow0, gsrc0, growr0, sem0), (grow1, gsrc1, growr1, sem1))

        def snap_and_fire(p):
            growb, gsrcb, growrb, semb = par[p]
            for q in range(FLUSH // 16):
                gsrcb[pl.ds(q * 16, 16)] = srcc[pl.ds(q * 16, 16)]
                growrb[pl.ds(q * 16, 16)] = rowc[pl.ds(q * 16, 16)]
            pltpu.async_copy(m_hbm.at[gsrcb.at[pl.ds(0, FLUSH)]], growb, semb)

        def acc_flush(p):
            growb, gsrcb, growrb, semb = par[p]
            pltpu.make_async_copy(m_hbm.at[gsrcb.at[pl.ds(0, FLUSH)]], growb,
                                  semb).wait()

            def grp(g, _):
                rv = growrb[pl.ds(g * 16, 16)]
                for e in range(16):
                    r = rv[e]
                    ge = g * 16 + e
                    for l in range(D_OUT // 16):
                        plsc.addupdate(acc.at[r, pl.ds(l * 16, 16)],
                                       growb[ge, pl.ds(l * 16, 16)])
                    plsc.addupdate(cacc.at[r], onev)
                return 0
            lax.fori_loop(0, FLUSH // 16, grp, 0)

        def flush(fk):
            @pl.when(fk >= 1)
            def _():
                lax.cond((fk & 1) == 1, lambda: acc_flush(0),
                         lambda: acc_flush(1))
            lax.cond((fk & 1) == 0, lambda: snap_and_fire(0),
                     lambda: snap_and_fire(1))
            srcc[pl.ds(0, 16)] = srcc[pl.ds(FLUSH, 16)]
            rowc[pl.ds(0, 16)] = rowc[pl.ds(FLUSH, 16)]
            srcc[pl.ds(16, 16)] = srcc[pl.ds(FLUSH + 16, 16)]
            rowc[pl.ds(16, 16)] = rowc[pl.ds(FLUSH + 16, 16)]

        NSTAGE = E_PAD // STAGE
        bufs = ((dstage0, sstage0, stsem0), (dstage1, sstage1, stsem1))

        def issue(t, b):
            db, sb, sm = bufs[b]
            pltpu.async_copy(dst_hbm.at[pl.ds(t * STAGE, STAGE)], db, sm)
            pltpu.async_copy(src_hbm.at[pl.ds(t * STAGE, STAGE)], sb, sm)

        issue(0, 0)
        issue(1, 1)

        def make_step(b):
            dstage, sstage, stsem = bufs[b]

            def stage_step(t, carry):
                pltpu.make_async_copy(
                    dst_hbm.at[pl.ds(t * STAGE, STAGE)], dstage, stsem).wait()
                pltpu.make_async_copy(
                    src_hbm.at[pl.ds(t * STAGE, STAGE)], sstage, stsem).wait()

                def group(g, carry):
                    off, fk = carry
                    d1 = dstage[pl.ds(g * 32, 16)]
                    sv1 = sstage[pl.ds(g * 32, 16)]
                    d2 = dstage[pl.ds(g * 32 + 16, 16)]
                    sv2 = sstage[pl.ds(g * 32 + 16, 16)]
                    rel1 = d1 - lo
                    rel2 = d2 - lo
                    ok1 = (rel1 >= 0) & (rel1 < BUCKET)
                    ok2 = (rel2 >= 0) & (rel2 < BUCKET)
                    pc1 = plsc.all_reduce_population_count(ok1)[0]
                    pc2 = plsc.all_reduce_population_count(ok2)[0]

                    def do_compact():
                        oki1 = jnp.where(ok1, 1, 0)
                        oki2 = jnp.where(ok2, 1, 0)
                        inc1 = plsc.cumsum(oki1)
                        inc2 = plsc.cumsum(oki2)
                        pos1 = jnp.where(ok1, off + inc1 - oki1, DUMP + iota16)
                        pos2 = jnp.where(ok2, off + pc1 + inc2 - oki2,
                                         DUMP + iota16)
                        plsc.store_scatter(srcc, [pos1], sv1)
                        plsc.store_scatter(rowc, [pos1], rel1)
                        plsc.store_scatter(srcc, [pos2], sv2)
                        plsc.store_scatter(rowc, [pos2], rel2)
                        return off + pc1 + pc2
                    off = lax.cond(pc1 + pc2 > 0, do_compact, lambda: off)

                    def do_flush():
                        flush(fk)
                        return off - FLUSH, fk + 1
                    return lax.cond(off >= FLUSH, do_flush, lambda: (off, fk))
                carry = lax.fori_loop(0, STAGE // 32, group, carry)
                nxt = t + 2

                @pl.when(nxt < NSTAGE)
                def _():
                    issue(nxt, b)
                return carry
            return stage_step

        step_for = (make_step(0), make_step(1))

        def pair_step(jo, carry):
            for b in range(2):
                carry = step_for[b](jo * 2 + b, carry)
            return carry

        off, fk = lax.fori_loop(0, NSTAGE // 2, pair_step, (0, 0))

        zi = jnp.zeros((16,), jnp.int32)
        gv = jnp.full((16,), GARBAGE, jnp.int32)
        for k in range(FLUSH // 16):
            srcc[pl.ds(off + k * 16, 16)] = zi
            rowc[pl.ds(off + k * 16, 16)] = gv
        flush(fk)
        fk = fk + 1
        lax.cond((fk & 1) == 1, lambda: acc_flush(0), lambda: acc_flush(1))

        for j in range(BUCKET // 16):
            cv = plsc.load_gather(cacc, [iota16 + j * 16, zi])
            acc[CNT_ROW + (j * 16) // D_OUT,
                pl.ds((j * 16) % D_OUT, 16)] = cv.astype(jnp.float32)

        pltpu.sync_copy(acc, out_hbm.at[c, s])

    return pl.kernel(body, out_type=out_type, mesh=mesh, scratch_types=scratch,
                     compiler_params=pltpu.CompilerParams(
                         needs_layout_passes=False,
                         use_tc_tiling_on_sc=False))


def kernel(x, edge_index, W1a, W2a, ln1_g, ln1_b, Wi, bi, W1b, W2b, ln2_g, ln2_b):
    src = edge_index[0].astype(jnp.int32)
    dst = edge_index[1].astype(jnp.int32)
    pad = E_PAD - E
    src_f = jnp.concatenate([src, jnp.zeros((pad,), jnp.int32)])
    dst_f = jnp.concatenate([dst, jnp.full((pad,), NW * BUCKET, jnp.int32)])

    bi2 = bi.reshape(1, D_OUT)
    g1 = ln1_g.reshape(1, D_OUT)
    b1 = ln1_b.reshape(1, D_OUT)
    g2 = ln2_g.reshape(1, D_OUT)
    b2 = ln2_b.reshape(1, D_OUT)

    def unpack(o):
        flat = o.reshape(NW, ACC_R * D_OUT)
        sums = flat[:, :BUCKET * D_OUT].reshape(NW * BUCKET, D_OUT)[:N]
        cnt = flat[:, CNT_ROW * D_OUT:CNT_ROW * D_OUT + BUCKET]
        cnt = cnt.reshape(NW * BUCKET)[:N]
        return sums, cnt

    m1 = _node_mlp(x, W1a, W2a)
    s1, cnt = unpack(_make_segsum()(m1, src_f, dst_f))
    cnt_b = jnp.broadcast_to(cnt[:, None], (N, D_OUT))
    h2, m2 = _mid_stage(s1, cnt_b, x, Wi, bi2, W1b, W2b, g1, b1)
    s2, _ = unpack(_make_segsum()(m2, src_f, dst_f))
    return _final_stage(s2, cnt_b, h2, g2, b2)


# speedup vs baseline: 1.4737x; 1.0753x over previous
import functools

import jax
import jax.numpy as jnp
from jax import lax
from jax.experimental import pallas as pl
from jax.experimental.pallas import tpu as pltpu
from jax.experimental.pallas import tpu_sc as plsc

N, E = 10000, 160000
D_IN, D_INNER, D_OUT = 256, 512, 256
NC, NS = 2, 16
NW = NC * NS
BUCKET = 320
GARBAGE = BUCKET
CNT_ROW = BUCKET + 1
ACC_R = 328
STAGE = 1024
E_PAD = 163840
FLUSH = 64
DUMP = 2 * FLUSH
RB = 1000


def _mlp_body(x_ref, w1_ref, w2_ref, o_ref):
    h = jnp.maximum(
        jnp.dot(x_ref[...], w1_ref[...], preferred_element_type=jnp.float32), 0.0)
    o_ref[...] = jnp.dot(h, w2_ref[...], preferred_element_type=jnp.float32)


def _node_mlp(xx, w1, w2):
    return pl.pallas_call(
        _mlp_body,
        grid=(N // RB,),
        in_specs=[
            pl.BlockSpec((RB, D_OUT), lambda i: (i, 0)),
            pl.BlockSpec((D_OUT, D_INNER), lambda i: (0, 0)),
            pl.BlockSpec((D_INNER, D_OUT), lambda i: (0, 0)),
        ],
        out_specs=pl.BlockSpec((RB, D_OUT), lambda i: (i, 0)),
        out_shape=jax.ShapeDtypeStruct((N, D_OUT), jnp.float32),
    )(xx, w1, w2)


def _mid_body(s_ref, c_ref, x_ref, wi_ref, bi_ref, w1_ref, w2_ref, g_ref, b_ref,
              h2_ref, m2_ref):
    cnt = c_ref[...]
    mean = s_ref[...] / jnp.maximum(cnt, 1.0)
    h = jnp.where(cnt > 0.0, mean, x_ref[...])
    mu = jnp.mean(h, axis=-1, keepdims=True)
    var = jnp.mean((h - mu) ** 2, axis=-1, keepdims=True)
    h = (h - mu) * lax.rsqrt(var + 1e-5) * g_ref[...] + b_ref[...]
    h2 = jnp.maximum(
        jnp.dot(h, wi_ref[...], preferred_element_type=jnp.float32) + bi_ref[...],
        0.0)
    h2_ref[...] = h2
    m2_ref[...] = jnp.dot(
        jnp.maximum(jnp.dot(h2, w1_ref[...], preferred_element_type=jnp.float32),
                    0.0),
        w2_ref[...], preferred_element_type=jnp.float32)


def _mid_stage(s1, cnt_b, x, wi, bi2, w1b, w2b, g1, b1):
    blk = pl.BlockSpec((RB, D_OUT), lambda i: (i, 0))
    full = lambda shape: pl.BlockSpec(shape, lambda i: (0,) * len(shape))
    return pl.pallas_call(
        _mid_body,
        grid=(N // RB,),
        in_specs=[
            blk, blk, blk,
            full((D_OUT, D_OUT)), full((1, D_OUT)),
            full((D_OUT, D_INNER)), full((D_INNER, D_OUT)),
            full((1, D_OUT)), full((1, D_OUT)),
        ],
        out_specs=[blk, blk],
        out_shape=[jax.ShapeDtypeStruct((N, D_OUT), jnp.float32),
                   jax.ShapeDtypeStruct((N, D_OUT), jnp.float32)],
    )(s1, cnt_b, x, wi, bi2, w1b, w2b, g1, b1)


def _final_body(s_ref, c_ref, h_ref, g_ref, b_ref, o_ref):
    cnt = c_ref[...]
    mean = s_ref[...] / jnp.maximum(cnt, 1.0)
    h = jnp.where(cnt > 0.0, mean, h_ref[...])
    mu = jnp.mean(h, axis=-1, keepdims=True)
    var = jnp.mean((h - mu) ** 2, axis=-1, keepdims=True)
    o_ref[...] = (h - mu) * lax.rsqrt(var + 1e-5) * g_ref[...] + b_ref[...]


def _final_stage(s2, cnt_b, h2, g2, b2):
    blk = pl.BlockSpec((RB, D_OUT), lambda i: (i, 0))
    full = lambda shape: pl.BlockSpec(shape, lambda i: (0,) * len(shape))
    return pl.pallas_call(
        _final_body,
        grid=(N // RB,),
        in_specs=[blk, blk, blk, full((1, D_OUT)), full((1, D_OUT))],
        out_specs=blk,
        out_shape=jax.ShapeDtypeStruct((N, D_OUT), jnp.float32),
    )(s2, cnt_b, h2, g2, b2)


@functools.cache
def _make_segsum():
    mesh = plsc.VectorSubcoreMesh(core_axis_name="c", subcore_axis_name="s")
    out_type = jax.ShapeDtypeStruct((NC, NS, ACC_R, D_OUT), jnp.float32)
    scratch = [
        pltpu.VMEM((ACC_R, D_OUT), jnp.float32),
        pltpu.VMEM((BUCKET + 1, 16), jnp.float32),
        pltpu.VMEM((FLUSH, D_OUT), jnp.float32),
        pltpu.VMEM((FLUSH, D_OUT), jnp.float32),
        pltpu.VMEM((STAGE,), jnp.int32),
        pltpu.VMEM((STAGE,), jnp.int32),
        pltpu.VMEM((STAGE,), jnp.int32),
        pltpu.VMEM((STAGE,), jnp.int32),
        pltpu.VMEM((2 * FLUSH + 16,), jnp.int32),
        pltpu.VMEM((2 * FLUSH + 16,), jnp.int32),
        pltpu.VMEM((FLUSH,), jnp.int32),
        pltpu.VMEM((FLUSH,), jnp.int32),
        pltpu.VMEM((FLUSH,), jnp.int32),
        pltpu.VMEM((FLUSH,), jnp.int32),
        pltpu.SemaphoreType.DMA,
        pltpu.SemaphoreType.DMA,
        pltpu.SemaphoreType.DMA,
        pltpu.SemaphoreType.DMA,
    ]

    def body(m_hbm, src_hbm, dst_hbm, out_hbm, acc, cacc, grow0, grow1,
             dstage0, sstage0, dstage1, sstage1, srcc, rowc,
             gsrc0, gsrc1, growr0, growr1, sem0, sem1, stsem0, stsem1):
        c = lax.axis_index("c")
        s = lax.axis_index("s")
        lo = (c * NS + s) * BUCKET
        zf = jnp.zeros((16,), jnp.float32)
        iota16 = lax.iota(jnp.int32, 16)
        onev = jnp.where(iota16 == 0, jnp.float32(1.0), jnp.float32(0.0))

        def zr(r, _):
            def zc(l, _):
                acc[r, pl.ds(l * 16, 16)] = zf
                return 0
            return lax.fori_loop(0, D_OUT // 16, zc, 0)
        lax.fori_loop(0, ACC_R, zr, 0)

        def zo(r, _):
            cacc[r, :] = zf
            return 0
        lax.fori_loop(0, BUCKET + 1, zo, 0)

        par = ((grow0, gsrc0, growr0, sem0), (grow1, gsrc1, growr1, sem1))

        def snap_and_fire(p):
            growb, gsrcb, growrb, semb = par[p]
            for q in range(FLUSH // 16):
                gsrcb[pl.ds(q * 16, 16)] = srcc[pl.ds(q * 16, 16)]
                growrb[pl.ds(q * 16, 16)] = rowc[pl.ds(q * 16, 16)]
            pltpu.async_copy(m_hbm.at[gsrcb.at[pl.ds(0, FLUSH)]], growb, semb)

        def acc_flush(p):
            growb, gsrcb, growrb, semb = par[p]
            pltpu.make_async_copy(m_hbm.at[gsrcb.at[pl.ds(0, FLUSH)]], growb,
                                  semb).wait()

            def grp(g, _):
                rv = growrb[pl.ds(g * 16, 16)]
                for e in range(16):
                    r = rv[e]
                    ge = g * 16 + e
                    for l in range(D_OUT // 16):
                        plsc.addupdate(acc.at[r, pl.ds(l * 16, 16)],
                                       growb[ge, pl.ds(l * 16, 16)])
                    plsc.addupdate(cacc.at[r], onev)
                return 0
            lax.fori_loop(0, FLUSH // 16, grp, 0)

        def flush(fk):
            @pl.when(fk >= 1)
            def _():
                lax.cond((fk & 1) == 1, lambda: acc_flush(0),
                         lambda: acc_flush(1))
            lax.cond((fk & 1) == 0, lambda: snap_and_fire(0),
                     lambda: snap_and_fire(1))
            for q in range(4):
                srcc[pl.ds(16 * q, 16)] = srcc[pl.ds(FLUSH + 16 * q, 16)]
                rowc[pl.ds(16 * q, 16)] = rowc[pl.ds(FLUSH + 16 * q, 16)]

        NSTAGE = E_PAD // STAGE
        bufs = ((dstage0, sstage0, stsem0), (dstage1, sstage1, stsem1))

        def issue(t, b):
            db, sb, sm = bufs[b]
            pltpu.async_copy(dst_hbm.at[pl.ds(t * STAGE, STAGE)], db, sm)
            pltpu.async_copy(src_hbm.at[pl.ds(t * STAGE, STAGE)], sb, sm)

        issue(0, 0)
        issue(1, 1)

        def make_step(b):
            dstage, sstage, stsem = bufs[b]

            def stage_step(t, carry):
                pltpu.make_async_copy(
                    dst_hbm.at[pl.ds(t * STAGE, STAGE)], dstage, stsem).wait()
                pltpu.make_async_copy(
                    src_hbm.at[pl.ds(t * STAGE, STAGE)], sstage, stsem).wait()

                def group(g, carry):
                    off, fk = carry
                    ds_ = [dstage[pl.ds(g * 64 + 16 * q, 16)] for q in range(4)]
                    svs = [sstage[pl.ds(g * 64 + 16 * q, 16)] for q in range(4)]
                    rels = [d - lo for d in ds_]
                    oks = [(r >= 0) & (r < BUCKET) for r in rels]
                    pcs = [plsc.all_reduce_population_count(o)[0] for o in oks]
                    tot = pcs[0] + pcs[1] + pcs[2] + pcs[3]

                    def do_compact():
                        base = off
                        for q in range(4):
                            oki = jnp.where(oks[q], 1, 0)
                            inc = plsc.cumsum(oki)
                            pos = jnp.where(oks[q], base + inc - oki,
                                            DUMP + iota16)
                            plsc.store_scatter(srcc, [pos], svs[q])
                            plsc.store_scatter(rowc, [pos], rels[q])
                            base = base + pcs[q]
                        return base
                    off = lax.cond(tot > 0, do_compact, lambda: off)

                    def do_flush():
                        flush(fk)
                        return off - FLUSH, fk + 1
                    return lax.cond(off >= FLUSH, do_flush, lambda: (off, fk))
                carry = lax.fori_loop(0, STAGE // 64, group, carry)
                nxt = t + 2

                @pl.when(nxt < NSTAGE)
                def _():
                    issue(nxt, b)
                return carry
            return stage_step

        step_for = (make_step(0), make_step(1))

        def pair_step(jo, carry):
            for b in range(2):
                carry = step_for[b](jo * 2 + b, carry)
            return carry

        off, fk = lax.fori_loop(0, NSTAGE // 2, pair_step, (0, 0))

        zi = jnp.zeros((16,), jnp.int32)
        gv = jnp.full((16,), GARBAGE, jnp.int32)
        for k in range(FLUSH // 16):
            srcc[pl.ds(off + k * 16, 16)] = zi
            rowc[pl.ds(off + k * 16, 16)] = gv
        flush(fk)
        fk = fk + 1
        lax.cond((fk & 1) == 1, lambda: acc_flush(0), lambda: acc_flush(1))

        for j in range(BUCKET // 16):
            cv = plsc.load_gather(cacc, [iota16 + j * 16, zi])
            acc[CNT_ROW + (j * 16) // D_OUT,
                pl.ds((j * 16) % D_OUT, 16)] = cv.astype(jnp.float32)

        pltpu.sync_copy(acc, out_hbm.at[c, s])

    return pl.kernel(body, out_type=out_type, mesh=mesh, scratch_types=scratch,
                     compiler_params=pltpu.CompilerParams(
                         needs_layout_passes=False,
                         use_tc_tiling_on_sc=False))


def kernel(x, edge_index, W1a, W2a, ln1_g, ln1_b, Wi, bi, W1b, W2b, ln2_g, ln2_b):
    src = edge_index[0].astype(jnp.int32)
    dst = edge_index[1].astype(jnp.int32)
    pad = E_PAD - E
    src_f = jnp.concatenate([src, jnp.zeros((pad,), jnp.int32)])
    dst_f = jnp.concatenate([dst, jnp.full((pad,), NW * BUCKET, jnp.int32)])

    bi2 = bi.reshape(1, D_OUT)
    g1 = ln1_g.reshape(1, D_OUT)
    b1 = ln1_b.reshape(1, D_OUT)
    g2 = ln2_g.reshape(1, D_OUT)
    b2 = ln2_b.reshape(1, D_OUT)

    def unpack(o):
        flat = o.reshape(NW, ACC_R * D_OUT)
        sums = flat[:, :BUCKET * D_OUT].reshape(NW * BUCKET, D_OUT)[:N]
        cnt = flat[:, CNT_ROW * D_OUT:CNT_ROW * D_OUT + BUCKET]
        cnt = cnt.reshape(NW * BUCKET)[:N]
        return sums, cnt

    m1 = _node_mlp(x, W1a, W2a)
    s1, cnt = unpack(_make_segsum()(m1, src_f, dst_f))
    cnt_b = jnp.broadcast_to(cnt[:, None], (N, D_OUT))
    h2, m2 = _mid_stage(s1, cnt_b, x, Wi, bi2, W1b, W2b, g1, b1)
    s2, _ = unpack(_make_segsum()(m2, src_f, dst_f))
    return _final_stage(s2, cnt_b, h2, g2, b2)


# speedup vs baseline: 1.4912x; 1.0119x over previous
import functools

import jax
import jax.numpy as jnp
from jax import lax
from jax.experimental import pallas as pl
from jax.experimental.pallas import tpu as pltpu
from jax.experimental.pallas import tpu_sc as plsc

N, E = 10000, 160000
D_IN, D_INNER, D_OUT = 256, 512, 256
NC, NS = 2, 16
NW = NC * NS
BUCKET = 320
GARBAGE = BUCKET
CNT_ROW = BUCKET + 1
ACC_R = 328
STAGE = 1600
FLUSH = 64
DUMP = 2 * FLUSH
RB = 1000


def _mlp_body(x_ref, w1_ref, w2_ref, o_ref):
    h = jnp.maximum(
        jnp.dot(x_ref[...], w1_ref[...], preferred_element_type=jnp.float32), 0.0)
    o_ref[...] = jnp.dot(h, w2_ref[...], preferred_element_type=jnp.float32)


def _node_mlp(xx, w1, w2):
    return pl.pallas_call(
        _mlp_body,
        grid=(N // RB,),
        in_specs=[
            pl.BlockSpec((RB, D_OUT), lambda i: (i, 0)),
            pl.BlockSpec((D_OUT, D_INNER), lambda i: (0, 0)),
            pl.BlockSpec((D_INNER, D_OUT), lambda i: (0, 0)),
        ],
        out_specs=pl.BlockSpec((RB, D_OUT), lambda i: (i, 0)),
        out_shape=jax.ShapeDtypeStruct((N, D_OUT), jnp.float32),
    )(xx, w1, w2)


def _mid_body(s_ref, c_ref, x_ref, wi_ref, bi_ref, w1_ref, w2_ref, g_ref, b_ref,
              h2_ref, m2_ref):
    cnt = c_ref[...]
    mean = s_ref[...] / jnp.maximum(cnt, 1.0)
    h = jnp.where(cnt > 0.0, mean, x_ref[...])
    mu = jnp.mean(h, axis=-1, keepdims=True)
    var = jnp.mean((h - mu) ** 2, axis=-1, keepdims=True)
    h = (h - mu) * lax.rsqrt(var + 1e-5) * g_ref[...] + b_ref[...]
    h2 = jnp.maximum(
        jnp.dot(h, wi_ref[...], preferred_element_type=jnp.float32) + bi_ref[...],
        0.0)
    h2_ref[...] = h2
    m2_ref[...] = jnp.dot(
        jnp.maximum(jnp.dot(h2, w1_ref[...], preferred_element_type=jnp.float32),
                    0.0),
        w2_ref[...], preferred_element_type=jnp.float32)


def _mid_stage(s1, cnt_b, x, wi, bi2, w1b, w2b, g1, b1):
    blk = pl.BlockSpec((RB, D_OUT), lambda i: (i, 0))
    full = lambda shape: pl.BlockSpec(shape, lambda i: (0,) * len(shape))
    return pl.pallas_call(
        _mid_body,
        grid=(N // RB,),
        in_specs=[
            blk, blk, blk,
            full((D_OUT, D_OUT)), full((1, D_OUT)),
            full((D_OUT, D_INNER)), full((D_INNER, D_OUT)),
            full((1, D_OUT)), full((1, D_OUT)),
        ],
        out_specs=[blk, blk],
        out_shape=[jax.ShapeDtypeStruct((N, D_OUT), jnp.float32),
                   jax.ShapeDtypeStruct((N, D_OUT), jnp.float32)],
    )(s1, cnt_b, x, wi, bi2, w1b, w2b, g1, b1)


def _final_body(s_ref, c_ref, h_ref, g_ref, b_ref, o_ref):
    cnt = c_ref[...]
    mean = s_ref[...] / jnp.maximum(cnt, 1.0)
    h = jnp.where(cnt > 0.0, mean, h_ref[...])
    mu = jnp.mean(h, axis=-1, keepdims=True)
    var = jnp.mean((h - mu) ** 2, axis=-1, keepdims=True)
    o_ref[...] = (h - mu) * lax.rsqrt(var + 1e-5) * g_ref[...] + b_ref[...]


def _final_stage(s2, cnt_b, h2, g2, b2):
    blk = pl.BlockSpec((RB, D_OUT), lambda i: (i, 0))
    full = lambda shape: pl.BlockSpec(shape, lambda i: (0,) * len(shape))
    return pl.pallas_call(
        _final_body,
        grid=(N // RB,),
        in_specs=[blk, blk, blk, full((1, D_OUT)), full((1, D_OUT))],
        out_specs=blk,
        out_shape=jax.ShapeDtypeStruct((N, D_OUT), jnp.float32),
    )(s2, cnt_b, h2, g2, b2)


@functools.cache
def _make_segsum():
    mesh = plsc.VectorSubcoreMesh(core_axis_name="c", subcore_axis_name="s")
    out_type = jax.ShapeDtypeStruct((NC, NS, ACC_R, D_OUT), jnp.float32)
    scratch = [
        pltpu.VMEM((ACC_R, D_OUT), jnp.float32),
        pltpu.VMEM((BUCKET + 1, 16), jnp.float32),
        pltpu.VMEM((FLUSH, D_OUT), jnp.float32),
        pltpu.VMEM((FLUSH, D_OUT), jnp.float32),
        pltpu.VMEM((STAGE,), jnp.int32),
        pltpu.VMEM((STAGE,), jnp.int32),
        pltpu.VMEM((STAGE,), jnp.int32),
        pltpu.VMEM((STAGE,), jnp.int32),
        pltpu.VMEM((2 * FLUSH + 16,), jnp.int32),
        pltpu.VMEM((2 * FLUSH + 16,), jnp.int32),
        pltpu.VMEM((FLUSH,), jnp.int32),
        pltpu.VMEM((FLUSH,), jnp.int32),
        pltpu.VMEM((FLUSH,), jnp.int32),
        pltpu.VMEM((FLUSH,), jnp.int32),
        pltpu.SemaphoreType.DMA,
        pltpu.SemaphoreType.DMA,
        pltpu.SemaphoreType.DMA,
        pltpu.SemaphoreType.DMA,
    ]

    def body(m_hbm, src_hbm, dst_hbm, out_hbm, acc, cacc, grow0, grow1,
             dstage0, sstage0, dstage1, sstage1, srcc, rowc,
             gsrc0, gsrc1, growr0, growr1, sem0, sem1, stsem0, stsem1):
        c = lax.axis_index("c")
        s = lax.axis_index("s")
        lo = (c * NS + s) * BUCKET
        zf = jnp.zeros((16,), jnp.float32)
        iota16 = lax.iota(jnp.int32, 16)
        onev = jnp.where(iota16 == 0, jnp.float32(1.0), jnp.float32(0.0))

        def zr(r, _):
            def zc(l, _):
                acc[r, pl.ds(l * 16, 16)] = zf
                return 0
            return lax.fori_loop(0, D_OUT // 16, zc, 0)
        lax.fori_loop(0, ACC_R, zr, 0)

        def zo(r, _):
            cacc[r, :] = zf
            return 0
        lax.fori_loop(0, BUCKET + 1, zo, 0)

        par = ((grow0, gsrc0, growr0, sem0), (grow1, gsrc1, growr1, sem1))

        def snap_and_fire(p):
            growb, gsrcb, growrb, semb = par[p]
            for q in range(FLUSH // 16):
                gsrcb[pl.ds(q * 16, 16)] = srcc[pl.ds(q * 16, 16)]
                growrb[pl.ds(q * 16, 16)] = rowc[pl.ds(q * 16, 16)]
            pltpu.async_copy(m_hbm.at[gsrcb.at[pl.ds(0, FLUSH)]], growb, semb)

        def acc_flush(p):
            growb, gsrcb, growrb, semb = par[p]
            pltpu.make_async_copy(m_hbm.at[gsrcb.at[pl.ds(0, FLUSH)]], growb,
                                  semb).wait()

            def grp(g, _):
                rv = growrb[pl.ds(g * 16, 16)]
                for e in range(16):
                    r = rv[e]
                    ge = g * 16 + e
                    for l in range(D_OUT // 16):
                        plsc.addupdate(acc.at[r, pl.ds(l * 16, 16)],
                                       growb[ge, pl.ds(l * 16, 16)])
                    plsc.addupdate(cacc.at[r], onev)
                return 0
            lax.fori_loop(0, FLUSH // 16, grp, 0)

        def flush(fk):
            @pl.when(fk >= 1)
            def _():
                lax.cond((fk & 1) == 1, lambda: acc_flush(0),
                         lambda: acc_flush(1))
            lax.cond((fk & 1) == 0, lambda: snap_and_fire(0),
                     lambda: snap_and_fire(1))
            for q in range(4):
                srcc[pl.ds(16 * q, 16)] = srcc[pl.ds(FLUSH + 16 * q, 16)]
                rowc[pl.ds(16 * q, 16)] = rowc[pl.ds(FLUSH + 16 * q, 16)]

        NSTAGE = E // STAGE
        bufs = ((dstage0, sstage0, stsem0), (dstage1, sstage1, stsem1))

        def issue(t, b):
            db, sb, sm = bufs[b]
            pltpu.async_copy(dst_hbm.at[pl.ds(t * STAGE, STAGE)], db, sm)
            pltpu.async_copy(src_hbm.at[pl.ds(t * STAGE, STAGE)], sb, sm)

        issue(0, 0)
        issue(1, 1)

        def make_step(b):
            dstage, sstage, stsem = bufs[b]

            def stage_step(t, carry):
                pltpu.make_async_copy(
                    dst_hbm.at[pl.ds(t * STAGE, STAGE)], dstage, stsem).wait()
                pltpu.make_async_copy(
                    src_hbm.at[pl.ds(t * STAGE, STAGE)], sstage, stsem).wait()

                def group(g, carry):
                    off, fk = carry
                    ds_ = [dstage[pl.ds(g * 64 + 16 * q, 16)] for q in range(4)]
                    svs = [sstage[pl.ds(g * 64 + 16 * q, 16)] for q in range(4)]
                    rels = [d - lo for d in ds_]
                    oks = [(r >= 0) & (r < BUCKET) for r in rels]
                    pcs = [plsc.all_reduce_population_count(o)[0] for o in oks]
                    tot = pcs[0] + pcs[1] + pcs[2] + pcs[3]

                    def do_compact():
                        base = off
                        for q in range(4):
                            oki = jnp.where(oks[q], 1, 0)
                            inc = plsc.cumsum(oki)
                            pos = jnp.where(oks[q], base + inc - oki,
                                            DUMP + iota16)
                            plsc.store_scatter(srcc, [pos], svs[q])
                            plsc.store_scatter(rowc, [pos], rels[q])
                            base = base + pcs[q]
                        return base
                    off = lax.cond(tot > 0, do_compact, lambda: off)

                    def do_flush():
                        flush(fk)
                        return off - FLUSH, fk + 1
                    return lax.cond(off >= FLUSH, do_flush, lambda: (off, fk))
                carry = lax.fori_loop(0, STAGE // 64, group, carry)
                nxt = t + 2

                @pl.when(nxt < NSTAGE)
                def _():
                    issue(nxt, b)
                return carry
            return stage_step

        step_for = (make_step(0), make_step(1))

        def pair_step(jo, carry):
            for b in range(2):
                carry = step_for[b](jo * 2 + b, carry)
            return carry

        off, fk = lax.fori_loop(0, NSTAGE // 2, pair_step, (0, 0))

        zi = jnp.zeros((16,), jnp.int32)
        gv = jnp.full((16,), GARBAGE, jnp.int32)
        for k in range(FLUSH // 16):
            srcc[pl.ds(off + k * 16, 16)] = zi
            rowc[pl.ds(off + k * 16, 16)] = gv
        flush(fk)
        fk = fk + 1
        lax.cond((fk & 1) == 1, lambda: acc_flush(0), lambda: acc_flush(1))

        for j in range(BUCKET // 16):
            cv = plsc.load_gather(cacc, [iota16 + j * 16, zi])
            acc[CNT_ROW + (j * 16) // D_OUT,
                pl.ds((j * 16) % D_OUT, 16)] = cv.astype(jnp.float32)

        pltpu.sync_copy(acc, out_hbm.at[c, s])

    return pl.kernel(body, out_type=out_type, mesh=mesh, scratch_types=scratch,
                     compiler_params=pltpu.CompilerParams(
                         needs_layout_passes=False,
                         use_tc_tiling_on_sc=False))


def kernel(x, edge_index, W1a, W2a, ln1_g, ln1_b, Wi, bi, W1b, W2b, ln2_g, ln2_b):
    src_f = edge_index[0].astype(jnp.int32)
    dst_f = edge_index[1].astype(jnp.int32)

    bi2 = bi.reshape(1, D_OUT)
    g1 = ln1_g.reshape(1, D_OUT)
    b1 = ln1_b.reshape(1, D_OUT)
    g2 = ln2_g.reshape(1, D_OUT)
    b2 = ln2_b.reshape(1, D_OUT)

    def unpack(o):
        flat = o.reshape(NW, ACC_R * D_OUT)
        sums = flat[:, :BUCKET * D_OUT].reshape(NW * BUCKET, D_OUT)[:N]
        cnt = flat[:, CNT_ROW * D_OUT:CNT_ROW * D_OUT + BUCKET]
        cnt = cnt.reshape(NW * BUCKET)[:N]
        return sums, cnt

    m1 = _node_mlp(x, W1a, W2a)
    s1, cnt = unpack(_make_segsum()(m1, src_f, dst_f))
    cnt_b = jnp.broadcast_to(cnt[:, None], (N, D_OUT))
    h2, m2 = _mid_stage(s1, cnt_b, x, Wi, bi2, W1b, W2b, g1, b1)
    s2, _ = unpack(_make_segsum()(m2, src_f, dst_f))
    return _final_stage(s2, cnt_b, h2, g2, b2)
